# baseline (speedup 1.0000x reference)
import jax
import jax.numpy as jnp
from jax.experimental import pallas as pl
from jax.experimental.pallas import tpu as pltpu

_N = 4096
_D = 64
_K = 4
_CH = 512
_NSTEP = _N // _CH
_NBUF = 4


def _copy(mask_hbm, buf, sem, chunk):
    return pltpu.make_async_copy(
        mask_hbm.at[pl.ds(chunk * _CH, _CH), :],
        buf,
        sem,
    )


def _mixed_op_body(mask_hbm, h_ref, w_ref, out_ref, b0, b1, b2, b3, hc_ref, sems):
    i = pl.program_id(0)
    bufs = (b0, b1, b2, b3)

    @pl.when(i == 0)
    def _prologue():
        for j in range(_NBUF):
            _copy(mask_hbm, bufs[j], sems.at[j], j).start()
        acc = jnp.zeros((_N, _D), jnp.float32)
        for k in range(_K):
            wk = w_ref[k]
            acc = acc + jnp.where(wk > 0, wk, 0.0) * h_ref[k]
        hc_ref[...] = acc.astype(jnp.bfloat16)

    c = jnp.float32(0.0)
    for k in range(_K):
        wk = w_ref[k]
        c = c + jnp.where(wk > 0, jnp.float32(0.0), wk)

    for s in range(_NBUF):

        @pl.when(i % _NBUF == s)
        def _run(s=s):
            _copy(mask_hbm, bufs[s], sems.at[s], i).wait()
            out_ref[...] = (
                jnp.dot(
                    bufs[s][...].astype(jnp.bfloat16),
                    hc_ref[...],
                    preferred_element_type=jnp.float32,
                )
                + c
            )
            nxt = i + _NBUF

            @pl.when(nxt < _NSTEP)
            def _prefetch():
                _copy(mask_hbm, bufs[s], sems.at[s], nxt).start()


@jax.jit
def kernel(mask_matrix, h_op_list, weights):
    return pl.pallas_call(
        _mixed_op_body,
        grid=(_NSTEP,),
        in_specs=[
            pl.BlockSpec(memory_space=pltpu.HBM),
            pl.BlockSpec((_K, _N, _D), lambda i: (0, 0, 0)),
            pl.BlockSpec(memory_space=pltpu.SMEM),
        ],
        out_specs=pl.BlockSpec((_CH, _D), lambda i: (i, 0)),
        out_shape=jax.ShapeDtypeStruct((_N, _D), jnp.float32),
        scratch_shapes=[
            pltpu.VMEM((_CH, _N), jnp.float32),
            pltpu.VMEM((_CH, _N), jnp.float32),
            pltpu.VMEM((_CH, _N), jnp.float32),
            pltpu.VMEM((_CH, _N), jnp.float32),
            pltpu.VMEM((_N, _D), jnp.bfloat16),
            pltpu.SemaphoreType.DMA((_NBUF,)),
        ],
    )(mask_matrix, h_op_list, weights)
